# Initial kernel scaffold; baseline (speedup 1.0000x reference)
import jax
import jax.numpy as jnp
from jax.experimental import pallas as pl


def kernel(x, edge_index, W1, b1, W2, b2, W3, b3):
    raise NotImplementedError("write your pallas kernel here")


# speedup vs baseline: 16.4516x; 16.4516x over previous
import functools

import jax
import jax.numpy as jnp
from jax import lax
from jax.experimental import pallas as pl
from jax.experimental.pallas import tpu as pltpu
from jax.experimental.pallas import tpu_sc as plsc

N = 10000
NP = 10240
E = 320000
D = 128
NC = 2
NS = 16
NW = NC * NS
CHUNK = 100
EPT = E // NW
CPT = EPT // CHUNK
RPS = NP // NS
ZROWS = 128

ROW_BLK = 2048

_mesh = plsc.VectorSubcoreMesh(core_axis_name="c", subcore_axis_name="s")


def _fill(buf, rows, value):
    @pl.loop(0, rows)
    def _(i):
        for j in range(D // 16):
            buf[i, pl.ds(j * 16, 16)] = jnp.full((16,), value, jnp.float32)


@functools.partial(
    pl.kernel,
    out_type=jax.ShapeDtypeStruct((NC, NP, D), jnp.float32),
    mesh=_mesh,
    scratch_types=[
        pltpu.VMEM((CPT, CHUNK), jnp.int32),
        pltpu.VMEM((CHUNK, D), jnp.float32),
        pltpu.VMEM_SHARED((NP, D), jnp.float32),
    ],
)
def _sc_degree(dst_hbm, out_hbm, dst_v, buf, acc):
    c = lax.axis_index("c")
    s = lax.axis_index("s")
    wid = c * NS + s
    _fill(buf, CHUNK, 0.0)
    for t in range(6):
        pltpu.sync_copy(buf.at[pl.ds(0, 96)],
                        acc.at[pl.ds(s * RPS + t * 96, 96)])
    pltpu.sync_copy(buf.at[pl.ds(0, 64)],
                    acc.at[pl.ds(s * RPS + 576, 64)])
    pltpu.sync_copy(dst_hbm.at[wid], dst_v)
    _fill(buf, CHUNK, 1.0)
    plsc.subcore_barrier()

    @pl.loop(0, CPT)
    def _(j):
        pltpu.sync_copy(buf.at[pl.ds(0, CHUNK)], acc.at[dst_v.at[j]], add=True)

    plsc.subcore_barrier()
    pltpu.sync_copy(acc.at[pl.ds(s * RPS, RPS)],
                    out_hbm.at[c].at[pl.ds(s * RPS, RPS)])


@functools.partial(
    pl.kernel,
    out_type=jax.ShapeDtypeStruct((NC, NP, D), jnp.float32),
    mesh=_mesh,
    scratch_types=[
        pltpu.VMEM((CPT, CHUNK), jnp.int32),
        pltpu.VMEM((CPT, CHUNK), jnp.int32),
        pltpu.VMEM((CHUNK, D), jnp.float32),
        pltpu.VMEM_SHARED((NP, D), jnp.float32),
    ],
)
def _sc_aggregate(hp_hbm, src_hbm, dst_hbm, out_hbm,
                  src_v, dst_v, rows, acc):
    c = lax.axis_index("c")
    s = lax.axis_index("s")
    wid = c * NS + s
    _fill(rows, CHUNK, 0.0)
    for t in range(6):
        pltpu.sync_copy(rows.at[pl.ds(0, 96)],
                        acc.at[pl.ds(s * RPS + t * 96, 96)])
    pltpu.sync_copy(rows.at[pl.ds(0, 64)],
                    acc.at[pl.ds(s * RPS + 576, 64)])
    pltpu.sync_copy(src_hbm.at[wid], src_v)
    pltpu.sync_copy(dst_hbm.at[wid], dst_v)
    plsc.subcore_barrier()

    @pl.loop(0, CPT)
    def _(j):
        pltpu.sync_copy(hp_hbm.at[src_v.at[j]], rows)
        pltpu.sync_copy(rows, acc.at[dst_v.at[j]], add=True)

    plsc.subcore_barrier()
    pltpu.sync_copy(acc.at[pl.ds(s * RPS, RPS)],
                    out_hbm.at[c].at[pl.ds(s * RPS, RPS)])


def _blk(i):
    return (i, 0)


def _rep(i):
    return (0, 0)


_node_spec = pl.BlockSpec((ROW_BLK, D), _blk)
_w_spec = pl.BlockSpec((D, D), _rep)
_b_spec = pl.BlockSpec((1, D), _rep)
_node_out = jax.ShapeDtypeStruct((NP, D), jnp.float32)


def _dot(a, b):
    return jax.lax.dot_general(a, b, (((1,), (0,)), ((), ())),
                               precision=jax.lax.Precision.HIGHEST,
                               preferred_element_type=jnp.float32)


def _mm1_body(x_ref, w_ref, o_ref):
    o_ref[...] = _dot(x_ref[...], w_ref[...])


def _prep_body(da_ref, db_ref, h_ref, dinv_ref, hp_ref):
    dinv = jax.lax.rsqrt(da_ref[...] + db_ref[...] + 1.0)
    dinv_ref[...] = dinv
    hp_ref[...] = h_ref[...] * dinv


def _layer_body(a0_ref, a1_ref, hp_ref, dinv_ref, b_ref, w_ref, o_ref):
    dinv = dinv_ref[...]
    y = dinv * (a0_ref[...] + a1_ref[...] + hp_ref[...]) + b_ref[...]
    y = jnp.maximum(y, 0.0)
    o_ref[...] = _dot(y, w_ref[...]) * dinv


def _final_body(a0_ref, a1_ref, hp_ref, dinv_ref, b_ref, o_ref):
    o_ref[...] = (dinv_ref[...] * (a0_ref[...] + a1_ref[...] + hp_ref[...])
                  + b_ref[...])


_GRID = NP // ROW_BLK

_tc_mm1 = pl.pallas_call(
    _mm1_body, grid=(_GRID,),
    in_specs=[_node_spec, _w_spec], out_specs=_node_spec,
    out_shape=_node_out)

_tc_prep = pl.pallas_call(
    _prep_body, grid=(_GRID,),
    in_specs=[_node_spec, _node_spec, _node_spec],
    out_specs=[_node_spec, _node_spec],
    out_shape=[_node_out, _node_out])

_tc_layer = pl.pallas_call(
    _layer_body, grid=(_GRID,),
    in_specs=[_node_spec, _node_spec, _node_spec, _node_spec, _b_spec, _w_spec],
    out_specs=_node_spec, out_shape=_node_out)

_tc_final = pl.pallas_call(
    _final_body, grid=(_GRID,),
    in_specs=[_node_spec, _node_spec, _node_spec, _node_spec, _b_spec],
    out_specs=_node_spec, out_shape=_node_out)


def kernel(x, edge_index, W1, b1, W2, b2, W3, b3):
    src = edge_index[0].reshape(NW, CPT, CHUNK)
    dst = edge_index[1].reshape(NW, CPT, CHUNK)
    x = jnp.pad(x, ((0, NP - N), (0, 0)))
    b1 = b1.reshape(1, D)
    b2 = b2.reshape(1, D)
    b3 = b3.reshape(1, D)

    deg = _sc_degree(dst)
    h1 = _tc_mm1(x, W1)
    dinv, hp1 = _tc_prep(deg[0], deg[1], h1)

    a = _sc_aggregate(hp1, src, dst)
    hp2 = _tc_layer(a[0], a[1], hp1, dinv, b1, W2)
    a = _sc_aggregate(hp2, src, dst)
    hp3 = _tc_layer(a[0], a[1], hp2, dinv, b2, W3)
    a = _sc_aggregate(hp3, src, dst)
    out = _tc_final(a[0], a[1], hp3, dinv, b3)
    return out[:N]
